# Initial kernel scaffold; baseline (speedup 1.0000x reference)
import jax
import jax.numpy as jnp
from jax.experimental import pallas as pl


def kernel(x, edge_index, eps, W1, b1, W2, b2, Wm, bm, Ws, bs):
    raise NotImplementedError("write your pallas kernel here")


# speedup vs baseline: 4.3222x; 4.3222x over previous
import functools

import jax
import jax.numpy as jnp
from jax import lax
from jax.experimental import pallas as pl
from jax.experimental.pallas import tpu as pltpu
from jax.experimental.pallas import tpu_sc as plsc

N = 10000
E = 160000
NC = 2
NS = 16
NW = NC * NS
CHUNK = 128
NCH = 40
EPT = NCH * CHUNK
EPAD = NW * EPT
NPAD = 10240
RPT = NPAD // NS

_MESH = plsc.VectorSubcoreMesh(core_axis_name="c", subcore_axis_name="s")
_SC_PARAMS = pltpu.CompilerParams(use_tc_tiling_on_sc=False)


@functools.partial(
    pl.kernel,
    out_type=jax.ShapeDtypeStruct((NC, NPAD, 2, 8), jnp.float32),
    mesh=_MESH,
    scratch_types=[
        pltpu.VMEM((NCH, CHUNK), jnp.int32),
        pltpu.VMEM((NCH, CHUNK), jnp.int32),
        pltpu.VMEM((CHUNK, 2, 8), jnp.float32),
        pltpu.VMEM((CHUNK, 2, 8), jnp.float32),
        pltpu.VMEM_SHARED((NPAD, 2, 8), jnp.float32),
    ],
    compiler_params=_SC_PARAMS,
)
def _sc_degrees(src_hbm, dst_hbm, ones_src_hbm, ones_dst_hbm, zeros_hbm,
                out_hbm, sidx, didx, vsrc, vdst, acc):
    cid = lax.axis_index("c")
    sid = lax.axis_index("s")
    w = cid * NS + sid
    r0 = sid * RPT
    pltpu.sync_copy(zeros_hbm.at[pl.ds(r0, RPT)], acc.at[pl.ds(r0, RPT)])
    pltpu.sync_copy(ones_src_hbm, vsrc)
    pltpu.sync_copy(ones_dst_hbm, vdst)
    pltpu.sync_copy(src_hbm.at[w], sidx)
    pltpu.sync_copy(dst_hbm.at[w], didx)
    plsc.subcore_barrier()

    def body(j, carry):
        pltpu.sync_copy(vsrc, acc.at[sidx.at[j]], add=True)
        pltpu.sync_copy(vdst, acc.at[didx.at[j]], add=True)
        return carry

    lax.fori_loop(0, NCH, body, 0)
    plsc.subcore_barrier()
    pltpu.sync_copy(acc.at[pl.ds(r0, RPT)], out_hbm.at[cid, pl.ds(r0, RPT)])


def _make_seg_sum(F):
    @functools.partial(
        pl.kernel,
        out_type=jax.ShapeDtypeStruct((NC, NPAD, F), jnp.float32),
        mesh=_MESH,
        scratch_types=[
            pltpu.VMEM((NCH, CHUNK), jnp.int32),
            pltpu.VMEM((NCH, CHUNK), jnp.int32),
            pltpu.VMEM((CHUNK, F), jnp.float32),
            pltpu.VMEM_SHARED((NPAD, F), jnp.float32),
            pltpu.SemaphoreType.DMA,
        ],
        compiler_params=_SC_PARAMS,
    )
    def seg_sum(t_hbm, src_hbm, dst_hbm, zeros_hbm, out_hbm,
                sidx, didx, rows, acc, sem):
        cid = lax.axis_index("c")
        sid = lax.axis_index("s")
        w = cid * NS + sid
        r0 = sid * RPT
        pltpu.sync_copy(zeros_hbm.at[pl.ds(r0, RPT)], acc.at[pl.ds(r0, RPT)])
        pltpu.sync_copy(src_hbm.at[w], sidx)
        pltpu.sync_copy(dst_hbm.at[w], didx)
        plsc.subcore_barrier()

        def body(j, carry):
            pltpu.async_copy(t_hbm.at[sidx.at[j]], rows, sem).wait()
            pltpu.sync_copy(rows, acc.at[didx.at[j]], add=True)
            return carry

        lax.fori_loop(0, NCH, body, 0)
        plsc.subcore_barrier()
        pltpu.sync_copy(acc.at[pl.ds(r0, RPT)], out_hbm.at[cid, pl.ds(r0, RPT)])

    return seg_sum


_seg_sum_64 = _make_seg_sum(64)
_seg_sum_32 = _make_seg_sum(32)


def _norm_body(degp_ref, out_ref):
    a = degp_ref[0] + degp_ref[1]
    d_src = jnp.sum(a[:, 0:8], axis=1, keepdims=True)
    d_dst = jnp.sum(a[:, 8:16], axis=1, keepdims=True)
    ns = lax.rsqrt(jnp.maximum(d_src, 1.0))
    nd = lax.rsqrt(jnp.maximum(d_dst, 1.0))
    out_ref[...] = jnp.concatenate([ns, nd], axis=1)


def _c1_body(x_ref, w_ref, nrm_ref, o_ref):
    ns = nrm_ref[:, 0:1]
    o_ref[...] = jnp.dot(x_ref[...] * ns, w_ref[...],
                         preferred_element_type=jnp.float32)


def _c2_body(agg_ref, nrm_ref, b_ref, w_ref, o_ref):
    s = agg_ref[0] + agg_ref[1]
    nd = nrm_ref[:, 1:2]
    ns = nrm_ref[:, 0:1]
    h = jnp.maximum(s * nd + b_ref[...], 0.0)
    o_ref[...] = jnp.dot(h * ns, w_ref[...], preferred_element_type=jnp.float32)


def _c3_body(agg_ref, nrm_ref, b_ref, o_ref):
    s = agg_ref[0] + agg_ref[1]
    nd = nrm_ref[:, 1:2]
    ns = nrm_ref[:, 0:1]
    o_ref[...] = (s * nd + b_ref[...]) * ns


def _c4_body(agg_ref, nrm_ref, wm_ref, bm_ref, ws_ref, bs_ref, eps_ref,
             zm_ref, zs_ref, z_ref):
    s = agg_ref[0] + agg_ref[1]
    nd = nrm_ref[:, 1:2]
    pre = s * nd
    zm = jnp.dot(pre, wm_ref[...], preferred_element_type=jnp.float32) + bm_ref[...]
    zs = jnp.dot(pre, ws_ref[...], preferred_element_type=jnp.float32) + bs_ref[...]
    zm_ref[...] = zm
    zs_ref[...] = zs
    z_ref[...] = zm + eps_ref[...] * zs


def _dec_body(zb_ref, zf_ref, o_ref):
    o_ref[...] = lax.dot_general(
        zb_ref[...], zf_ref[...],
        dimension_numbers=(((1,), (1,)), ((), ())),
        preferred_element_type=jnp.float32)


_RB = 1024
_DB = 200


def kernel(x, edge_index, eps, W1, b1, W2, b2, Wm, bm, Ws, bs):
    f32 = jnp.float32
    pad = jnp.full((EPAD - E,), N, jnp.int32)
    src = jnp.concatenate([edge_index[0], pad]).reshape(NW, NCH, CHUNK)
    dst = jnp.concatenate([edge_index[1], pad]).reshape(NW, NCH, CHUNK)

    ones_src = jnp.concatenate(
        [jnp.ones((CHUNK, 1, 8), f32), jnp.zeros((CHUNK, 1, 8), f32)], axis=1) / 8.0
    ones_dst = jnp.concatenate(
        [jnp.zeros((CHUNK, 1, 8), f32), jnp.ones((CHUNK, 1, 8), f32)], axis=1) / 8.0
    zeros_deg = jnp.zeros((NPAD, 2, 8), f32)
    zeros_64 = jnp.zeros((NPAD, 64), f32)
    zeros_32 = jnp.zeros((NPAD, 32), f32)

    degp = _sc_degrees(src, dst, ones_src, ones_dst, zeros_deg)
    degp = degp[:, :N].reshape(NC, N, 16)
    norms = pl.pallas_call(
        _norm_body,
        out_shape=jax.ShapeDtypeStruct((N, 2), f32),
    )(degp)

    grid = NPAD // _RB
    row_spec = lambda F: pl.BlockSpec((_RB, F), lambda i: (i, 0))
    agg_spec = lambda F: pl.BlockSpec((NC, _RB, F), lambda i: (0, i, 0))
    full = lambda shape: pl.BlockSpec(shape, lambda i: tuple(0 for _ in shape))

    t1 = pl.pallas_call(
        _c1_body,
        grid=(grid,),
        in_specs=[row_spec(128), full((128, 64)), row_spec(2)],
        out_specs=row_spec(64),
        out_shape=jax.ShapeDtypeStruct((NPAD, 64), f32),
    )(x, W1, norms)
    agg1 = _seg_sum_64(t1, src, dst, zeros_64)

    t2 = pl.pallas_call(
        _c2_body,
        grid=(grid,),
        in_specs=[agg_spec(64), row_spec(2), full((1, 64)), full((64, 32))],
        out_specs=row_spec(32),
        out_shape=jax.ShapeDtypeStruct((NPAD, 32), f32),
    )(agg1, norms, b1.reshape(1, 64), W2)
    agg2 = _seg_sum_32(t2, src, dst, zeros_32)

    t3 = pl.pallas_call(
        _c3_body,
        grid=(grid,),
        in_specs=[agg_spec(32), row_spec(2), full((1, 32))],
        out_specs=row_spec(32),
        out_shape=jax.ShapeDtypeStruct((NPAD, 32), f32),
    )(agg2, norms, b2.reshape(1, 32))
    agg3 = _seg_sum_32(t3, src, dst, zeros_32)

    zm_full, zs_full, z_full = pl.pallas_call(
        _c4_body,
        grid=(grid,),
        in_specs=[agg_spec(32), row_spec(2), full((32, 16)), full((1, 16)),
                  full((32, 16)), full((1, 16)), full((1, 16))],
        out_specs=[row_spec(16), row_spec(16), row_spec(16)],
        out_shape=[jax.ShapeDtypeStruct((NPAD, 16), f32)] * 3,
    )(agg3, norms, Wm, bm.reshape(1, 16), Ws, bs.reshape(1, 16),
      eps.reshape(1, 16))

    z_mean = zm_full[:N]
    z_log_std = zs_full[:N]
    z = z_full[:N]

    adj = pl.pallas_call(
        _dec_body,
        grid=(N // _DB,),
        in_specs=[pl.BlockSpec((_DB, 16), lambda i: (i, 0)),
                  pl.BlockSpec((N, 16), lambda i: (0, 0))],
        out_specs=pl.BlockSpec((_DB, N), lambda i: (i, 0)),
        out_shape=jax.ShapeDtypeStruct((N, N), f32),
    )(z, z)

    return (z_mean, z_log_std, adj)


# speedup vs baseline: 4.3405x; 1.0042x over previous
import functools

import jax
import jax.numpy as jnp
from jax import lax
from jax.experimental import pallas as pl
from jax.experimental.pallas import tpu as pltpu
from jax.experimental.pallas import tpu_sc as plsc

N = 10000
E = 160000
NC = 2
NS = 16
NW = NC * NS
CHUNK = 128
NCH = 40
EPT = NCH * CHUNK
EPAD = NW * EPT
NPAD = 10240
RPT = NPAD // NS

_MESH = plsc.VectorSubcoreMesh(core_axis_name="c", subcore_axis_name="s")
_SC_PARAMS = pltpu.CompilerParams(use_tc_tiling_on_sc=False)


@functools.partial(
    pl.kernel,
    out_type=jax.ShapeDtypeStruct((NC, NPAD, 2, 8), jnp.float32),
    mesh=_MESH,
    scratch_types=[
        pltpu.VMEM((NCH, CHUNK), jnp.int32),
        pltpu.VMEM((NCH, CHUNK), jnp.int32),
        pltpu.VMEM((CHUNK, 2, 8), jnp.float32),
        pltpu.VMEM((CHUNK, 2, 8), jnp.float32),
        pltpu.VMEM_SHARED((NPAD, 2, 8), jnp.float32),
    ],
    compiler_params=_SC_PARAMS,
)
def _sc_degrees(src_hbm, dst_hbm, ones_src_hbm, ones_dst_hbm, zeros_hbm,
                out_hbm, sidx, didx, vsrc, vdst, acc):
    cid = lax.axis_index("c")
    sid = lax.axis_index("s")
    w = cid * NS + sid
    r0 = sid * RPT
    pltpu.sync_copy(zeros_hbm.at[pl.ds(r0, RPT)], acc.at[pl.ds(r0, RPT)])
    pltpu.sync_copy(ones_src_hbm, vsrc)
    pltpu.sync_copy(ones_dst_hbm, vdst)
    pltpu.sync_copy(src_hbm.at[w], sidx)
    pltpu.sync_copy(dst_hbm.at[w], didx)
    plsc.subcore_barrier()

    def body(j, carry):
        pltpu.sync_copy(vsrc, acc.at[sidx.at[j]], add=True)
        pltpu.sync_copy(vdst, acc.at[didx.at[j]], add=True)
        return carry

    lax.fori_loop(0, NCH, body, 0)
    plsc.subcore_barrier()
    pltpu.sync_copy(acc.at[pl.ds(r0, RPT)], out_hbm.at[cid, pl.ds(r0, RPT)])


def _make_seg_sum(F):
    @functools.partial(
        pl.kernel,
        out_type=jax.ShapeDtypeStruct((NC, NPAD, F), jnp.float32),
        mesh=_MESH,
        scratch_types=[
            pltpu.VMEM((NCH, CHUNK), jnp.int32),
            pltpu.VMEM((NCH, CHUNK), jnp.int32),
            pltpu.VMEM((2, CHUNK, F), jnp.float32),
            pltpu.VMEM_SHARED((NPAD, F), jnp.float32),
            pltpu.SemaphoreType.DMA,
            pltpu.SemaphoreType.DMA,
        ],
        compiler_params=_SC_PARAMS,
    )
    def seg_sum(t_hbm, src_hbm, dst_hbm, zeros_hbm, out_hbm,
                sidx, didx, rows, acc, sem0, sem1):
        cid = lax.axis_index("c")
        sid = lax.axis_index("s")
        w = cid * NS + sid
        r0 = sid * RPT
        pltpu.sync_copy(zeros_hbm.at[pl.ds(r0, RPT)], acc.at[pl.ds(r0, RPT)])
        pltpu.sync_copy(src_hbm.at[w], sidx)
        pltpu.sync_copy(dst_hbm.at[w], didx)
        plsc.subcore_barrier()

        pltpu.async_copy(t_hbm.at[sidx.at[0]], rows.at[0], sem0)

        def body(i, carry):
            j = 2 * i
            pltpu.make_async_copy(t_hbm.at[sidx.at[j]], rows.at[0], sem0).wait()
            pltpu.async_copy(t_hbm.at[sidx.at[j + 1]], rows.at[1], sem1)
            pltpu.sync_copy(rows.at[0], acc.at[didx.at[j]], add=True)
            pltpu.make_async_copy(t_hbm.at[sidx.at[j + 1]], rows.at[1], sem1).wait()
            pltpu.async_copy(t_hbm.at[sidx.at[j + 2]], rows.at[0], sem0)
            pltpu.sync_copy(rows.at[1], acc.at[didx.at[j + 1]], add=True)
            return carry

        lax.fori_loop(0, NCH // 2 - 1, body, 0)
        j = NCH - 2
        pltpu.make_async_copy(t_hbm.at[sidx.at[j]], rows.at[0], sem0).wait()
        pltpu.async_copy(t_hbm.at[sidx.at[j + 1]], rows.at[1], sem1)
        pltpu.sync_copy(rows.at[0], acc.at[didx.at[j]], add=True)
        pltpu.make_async_copy(t_hbm.at[sidx.at[j + 1]], rows.at[1], sem1).wait()
        pltpu.sync_copy(rows.at[1], acc.at[didx.at[j + 1]], add=True)

        plsc.subcore_barrier()
        pltpu.sync_copy(acc.at[pl.ds(r0, RPT)], out_hbm.at[cid, pl.ds(r0, RPT)])

    return seg_sum


_seg_sum_64 = _make_seg_sum(64)
_seg_sum_32 = _make_seg_sum(32)


def _norm_body(degp_ref, out_ref):
    a = degp_ref[0] + degp_ref[1]
    d_src = jnp.sum(a[:, 0:8], axis=1, keepdims=True)
    d_dst = jnp.sum(a[:, 8:16], axis=1, keepdims=True)
    ns = lax.rsqrt(jnp.maximum(d_src, 1.0))
    nd = lax.rsqrt(jnp.maximum(d_dst, 1.0))
    out_ref[...] = jnp.concatenate([ns, nd], axis=1)


def _c1_body(x_ref, w_ref, nrm_ref, o_ref):
    ns = nrm_ref[:, 0:1]
    o_ref[...] = jnp.dot(x_ref[...] * ns, w_ref[...],
                         preferred_element_type=jnp.float32)


def _c2_body(agg_ref, nrm_ref, b_ref, w_ref, o_ref):
    s = agg_ref[0] + agg_ref[1]
    nd = nrm_ref[:, 1:2]
    ns = nrm_ref[:, 0:1]
    h = jnp.maximum(s * nd + b_ref[...], 0.0)
    o_ref[...] = jnp.dot(h * ns, w_ref[...], preferred_element_type=jnp.float32)


def _c3_body(agg_ref, nrm_ref, b_ref, o_ref):
    s = agg_ref[0] + agg_ref[1]
    nd = nrm_ref[:, 1:2]
    ns = nrm_ref[:, 0:1]
    o_ref[...] = (s * nd + b_ref[...]) * ns


def _c4_body(agg_ref, nrm_ref, wm_ref, bm_ref, ws_ref, bs_ref, eps_ref,
             zm_ref, zs_ref, z_ref):
    s = agg_ref[0] + agg_ref[1]
    nd = nrm_ref[:, 1:2]
    pre = s * nd
    zm = jnp.dot(pre, wm_ref[...], preferred_element_type=jnp.float32) + bm_ref[...]
    zs = jnp.dot(pre, ws_ref[...], preferred_element_type=jnp.float32) + bs_ref[...]
    zm_ref[...] = zm
    zs_ref[...] = zs
    z_ref[...] = zm + eps_ref[...] * zs


def _dec_body(zb_ref, zf_ref, o_ref):
    o_ref[...] = lax.dot_general(
        zb_ref[...], zf_ref[...],
        dimension_numbers=(((1,), (1,)), ((), ())),
        preferred_element_type=jnp.float32)


_RB = 1024
_DB = 200


def kernel(x, edge_index, eps, W1, b1, W2, b2, Wm, bm, Ws, bs):
    f32 = jnp.float32
    pad = jnp.full((EPAD - E,), N, jnp.int32)
    src = jnp.concatenate([edge_index[0], pad]).reshape(NW, NCH, CHUNK)
    dst = jnp.concatenate([edge_index[1], pad]).reshape(NW, NCH, CHUNK)

    ones_src = jnp.concatenate(
        [jnp.ones((CHUNK, 1, 8), f32), jnp.zeros((CHUNK, 1, 8), f32)], axis=1) / 8.0
    ones_dst = jnp.concatenate(
        [jnp.zeros((CHUNK, 1, 8), f32), jnp.ones((CHUNK, 1, 8), f32)], axis=1) / 8.0
    zeros_deg = jnp.zeros((NPAD, 2, 8), f32)
    zeros_64 = jnp.zeros((NPAD, 64), f32)
    zeros_32 = jnp.zeros((NPAD, 32), f32)

    degp = _sc_degrees(src, dst, ones_src, ones_dst, zeros_deg)
    degp = degp[:, :N].reshape(NC, N, 16)
    norms = pl.pallas_call(
        _norm_body,
        out_shape=jax.ShapeDtypeStruct((N, 2), f32),
    )(degp)

    grid = NPAD // _RB
    row_spec = lambda F: pl.BlockSpec((_RB, F), lambda i: (i, 0))
    agg_spec = lambda F: pl.BlockSpec((NC, _RB, F), lambda i: (0, i, 0))
    full = lambda shape: pl.BlockSpec(shape, lambda i: tuple(0 for _ in shape))

    t1 = pl.pallas_call(
        _c1_body,
        grid=(grid,),
        in_specs=[row_spec(128), full((128, 64)), row_spec(2)],
        out_specs=row_spec(64),
        out_shape=jax.ShapeDtypeStruct((NPAD, 64), f32),
    )(x, W1, norms)
    agg1 = _seg_sum_64(t1, src, dst, zeros_64)

    t2 = pl.pallas_call(
        _c2_body,
        grid=(grid,),
        in_specs=[agg_spec(64), row_spec(2), full((1, 64)), full((64, 32))],
        out_specs=row_spec(32),
        out_shape=jax.ShapeDtypeStruct((NPAD, 32), f32),
    )(agg1, norms, b1.reshape(1, 64), W2)
    agg2 = _seg_sum_32(t2, src, dst, zeros_32)

    t3 = pl.pallas_call(
        _c3_body,
        grid=(grid,),
        in_specs=[agg_spec(32), row_spec(2), full((1, 32))],
        out_specs=row_spec(32),
        out_shape=jax.ShapeDtypeStruct((NPAD, 32), f32),
    )(agg2, norms, b2.reshape(1, 32))
    agg3 = _seg_sum_32(t3, src, dst, zeros_32)

    zm_full, zs_full, z_full = pl.pallas_call(
        _c4_body,
        grid=(grid,),
        in_specs=[agg_spec(32), row_spec(2), full((32, 16)), full((1, 16)),
                  full((32, 16)), full((1, 16)), full((1, 16))],
        out_specs=[row_spec(16), row_spec(16), row_spec(16)],
        out_shape=[jax.ShapeDtypeStruct((NPAD, 16), f32)] * 3,
    )(agg3, norms, Wm, bm.reshape(1, 16), Ws, bs.reshape(1, 16),
      eps.reshape(1, 16))

    z_mean = zm_full[:N]
    z_log_std = zs_full[:N]
    z = z_full[:N]

    adj = pl.pallas_call(
        _dec_body,
        grid=(N // _DB,),
        in_specs=[pl.BlockSpec((_DB, 16), lambda i: (i, 0)),
                  pl.BlockSpec((N, 16), lambda i: (0, 0))],
        out_specs=pl.BlockSpec((_DB, N), lambda i: (i, 0)),
        out_shape=jax.ShapeDtypeStruct((N, N), f32),
    )(z, z)

    return (z_mean, z_log_std, adj)


# speedup vs baseline: 4.4402x; 1.0230x over previous
import functools

import jax
import jax.numpy as jnp
from jax import lax
from jax.experimental import pallas as pl
from jax.experimental.pallas import tpu as pltpu
from jax.experimental.pallas import tpu_sc as plsc

N = 10000
E = 160000
NC = 2
NS = 16
NW = NC * NS
CHUNK = 128
NCH = 40
EPT = NCH * CHUNK
EPAD = NW * EPT
NPAD = 10240
RPT = NPAD // NS

_MESH = plsc.VectorSubcoreMesh(core_axis_name="c", subcore_axis_name="s")
_SC_PARAMS = pltpu.CompilerParams(use_tc_tiling_on_sc=False)


@functools.partial(
    pl.kernel,
    out_type=jax.ShapeDtypeStruct((NC, NPAD, 2, 8), jnp.float32),
    mesh=_MESH,
    scratch_types=[
        pltpu.VMEM((NCH, CHUNK), jnp.int32),
        pltpu.VMEM((NCH, CHUNK), jnp.int32),
        pltpu.VMEM((CHUNK, 2, 8), jnp.float32),
        pltpu.VMEM((CHUNK, 2, 8), jnp.float32),
        pltpu.VMEM_SHARED((NPAD, 2, 8), jnp.float32),
        pltpu.SemaphoreType.DMA,
    ],
    compiler_params=_SC_PARAMS,
)
def _sc_degrees(src_hbm, dst_hbm, ones_src_hbm, ones_dst_hbm, zeros_hbm,
                out_hbm, sidx, didx, vsrc, vdst, acc, ssem):
    cid = lax.axis_index("c")
    sid = lax.axis_index("s")
    w = cid * NS + sid
    r0 = sid * RPT
    pltpu.sync_copy(zeros_hbm.at[pl.ds(r0, RPT)], acc.at[pl.ds(r0, RPT)])
    pltpu.sync_copy(ones_src_hbm, vsrc)
    pltpu.sync_copy(ones_dst_hbm, vdst)
    pltpu.sync_copy(src_hbm.at[w], sidx)
    pltpu.sync_copy(dst_hbm.at[w], didx)
    plsc.subcore_barrier()

    def body(g, carry):
        sds = ([pltpu.async_copy(vsrc, acc.at[sidx.at[8 * g + b]],
                                 ssem, add=True) for b in range(8)] +
               [pltpu.async_copy(vdst, acc.at[didx.at[8 * g + b]],
                                 ssem, add=True) for b in range(8)])
        for d in sds:
            d.wait()
        return carry

    lax.fori_loop(0, NCH // 8, body, 0)
    plsc.subcore_barrier()
    pltpu.sync_copy(acc.at[pl.ds(r0, RPT)], out_hbm.at[cid, pl.ds(r0, RPT)])


def _make_seg_sum(F):
    @functools.partial(
        pl.kernel,
        out_type=jax.ShapeDtypeStruct((NC, NPAD, F), jnp.float32),
        mesh=_MESH,
        scratch_types=[
            pltpu.VMEM((NCH, CHUNK), jnp.int32),
            pltpu.VMEM((NCH, CHUNK), jnp.int32),
            pltpu.VMEM((8, CHUNK, F), jnp.float32),
            pltpu.VMEM_SHARED((NPAD, F), jnp.float32),
            pltpu.SemaphoreType.DMA,
            pltpu.SemaphoreType.DMA,
        ],
        compiler_params=_SC_PARAMS,
    )
    def seg_sum(t_hbm, src_hbm, dst_hbm, zeros_hbm, out_hbm,
                sidx, didx, rows, acc, gsem, ssem):
        cid = lax.axis_index("c")
        sid = lax.axis_index("s")
        w = cid * NS + sid
        r0 = sid * RPT
        pltpu.sync_copy(zeros_hbm.at[pl.ds(r0, RPT)], acc.at[pl.ds(r0, RPT)])
        pltpu.sync_copy(src_hbm.at[w], sidx)
        pltpu.sync_copy(dst_hbm.at[w], didx)
        plsc.subcore_barrier()

        def body(g, carry):
            gds = [pltpu.async_copy(t_hbm.at[sidx.at[8 * g + b]],
                                    rows.at[b], gsem) for b in range(8)]
            for d in gds:
                d.wait()
            sds = [pltpu.async_copy(rows.at[b], acc.at[didx.at[8 * g + b]],
                                    ssem, add=True) for b in range(8)]
            for d in sds:
                d.wait()
            return carry

        lax.fori_loop(0, NCH // 8, body, 0)
        plsc.subcore_barrier()
        pltpu.sync_copy(acc.at[pl.ds(r0, RPT)], out_hbm.at[cid, pl.ds(r0, RPT)])

    return seg_sum


_seg_sum_64 = _make_seg_sum(64)
_seg_sum_32 = _make_seg_sum(32)


def _norm_body(degp_ref, out_ref):
    a = degp_ref[0] + degp_ref[1]
    d_src = jnp.sum(a[:, 0:8], axis=1, keepdims=True)
    d_dst = jnp.sum(a[:, 8:16], axis=1, keepdims=True)
    ns = lax.rsqrt(jnp.maximum(d_src, 1.0))
    nd = lax.rsqrt(jnp.maximum(d_dst, 1.0))
    out_ref[...] = jnp.concatenate([ns, nd], axis=1)


def _c1_body(x_ref, w_ref, nrm_ref, o_ref):
    ns = nrm_ref[:, 0:1]
    o_ref[...] = jnp.dot(x_ref[...] * ns, w_ref[...],
                         preferred_element_type=jnp.float32)


def _c2_body(agg_ref, nrm_ref, b_ref, w_ref, o_ref):
    s = agg_ref[0] + agg_ref[1]
    nd = nrm_ref[:, 1:2]
    ns = nrm_ref[:, 0:1]
    h = jnp.maximum(s * nd + b_ref[...], 0.0)
    o_ref[...] = jnp.dot(h * ns, w_ref[...], preferred_element_type=jnp.float32)


def _c3_body(agg_ref, nrm_ref, b_ref, o_ref):
    s = agg_ref[0] + agg_ref[1]
    nd = nrm_ref[:, 1:2]
    ns = nrm_ref[:, 0:1]
    o_ref[...] = (s * nd + b_ref[...]) * ns


def _c4_body(agg_ref, nrm_ref, wm_ref, bm_ref, ws_ref, bs_ref, eps_ref,
             zm_ref, zs_ref, z_ref):
    s = agg_ref[0] + agg_ref[1]
    nd = nrm_ref[:, 1:2]
    pre = s * nd
    zm = jnp.dot(pre, wm_ref[...], preferred_element_type=jnp.float32) + bm_ref[...]
    zs = jnp.dot(pre, ws_ref[...], preferred_element_type=jnp.float32) + bs_ref[...]
    zm_ref[...] = zm
    zs_ref[...] = zs
    z_ref[...] = zm + eps_ref[...] * zs


def _dec_body(zb_ref, zf_ref, o_ref):
    o_ref[...] = lax.dot_general(
        zb_ref[...], zf_ref[...],
        dimension_numbers=(((1,), (1,)), ((), ())),
        preferred_element_type=jnp.float32)


_RB = 1024
_DB = 200


def kernel(x, edge_index, eps, W1, b1, W2, b2, Wm, bm, Ws, bs):
    f32 = jnp.float32
    pad = jnp.full((EPAD - E,), N, jnp.int32)
    src = jnp.concatenate([edge_index[0], pad]).reshape(NW, NCH, CHUNK)
    dst = jnp.concatenate([edge_index[1], pad]).reshape(NW, NCH, CHUNK)

    ones_src = jnp.concatenate(
        [jnp.ones((CHUNK, 1, 8), f32), jnp.zeros((CHUNK, 1, 8), f32)], axis=1) / 8.0
    ones_dst = jnp.concatenate(
        [jnp.zeros((CHUNK, 1, 8), f32), jnp.ones((CHUNK, 1, 8), f32)], axis=1) / 8.0
    zeros_deg = jnp.zeros((NPAD, 2, 8), f32)
    zeros_64 = jnp.zeros((NPAD, 64), f32)
    zeros_32 = jnp.zeros((NPAD, 32), f32)

    degp = _sc_degrees(src, dst, ones_src, ones_dst, zeros_deg)
    degp = degp[:, :N].reshape(NC, N, 16)
    norms = pl.pallas_call(
        _norm_body,
        out_shape=jax.ShapeDtypeStruct((N, 2), f32),
    )(degp)

    grid = NPAD // _RB
    row_spec = lambda F: pl.BlockSpec((_RB, F), lambda i: (i, 0))
    agg_spec = lambda F: pl.BlockSpec((NC, _RB, F), lambda i: (0, i, 0))
    full = lambda shape: pl.BlockSpec(shape, lambda i: tuple(0 for _ in shape))

    t1 = pl.pallas_call(
        _c1_body,
        grid=(grid,),
        in_specs=[row_spec(128), full((128, 64)), row_spec(2)],
        out_specs=row_spec(64),
        out_shape=jax.ShapeDtypeStruct((NPAD, 64), f32),
    )(x, W1, norms)
    agg1 = _seg_sum_64(t1, src, dst, zeros_64)

    t2 = pl.pallas_call(
        _c2_body,
        grid=(grid,),
        in_specs=[agg_spec(64), row_spec(2), full((1, 64)), full((64, 32))],
        out_specs=row_spec(32),
        out_shape=jax.ShapeDtypeStruct((NPAD, 32), f32),
    )(agg1, norms, b1.reshape(1, 64), W2)
    agg2 = _seg_sum_32(t2, src, dst, zeros_32)

    t3 = pl.pallas_call(
        _c3_body,
        grid=(grid,),
        in_specs=[agg_spec(32), row_spec(2), full((1, 32))],
        out_specs=row_spec(32),
        out_shape=jax.ShapeDtypeStruct((NPAD, 32), f32),
    )(agg2, norms, b2.reshape(1, 32))
    agg3 = _seg_sum_32(t3, src, dst, zeros_32)

    zm_full, zs_full, z_full = pl.pallas_call(
        _c4_body,
        grid=(grid,),
        in_specs=[agg_spec(32), row_spec(2), full((32, 16)), full((1, 16)),
                  full((32, 16)), full((1, 16)), full((1, 16))],
        out_specs=[row_spec(16), row_spec(16), row_spec(16)],
        out_shape=[jax.ShapeDtypeStruct((NPAD, 16), f32)] * 3,
    )(agg3, norms, Wm, bm.reshape(1, 16), Ws, bs.reshape(1, 16),
      eps.reshape(1, 16))

    z_mean = zm_full[:N]
    z_log_std = zs_full[:N]
    z = z_full[:N]

    adj = pl.pallas_call(
        _dec_body,
        grid=(N // _DB,),
        in_specs=[pl.BlockSpec((_DB, 16), lambda i: (i, 0)),
                  pl.BlockSpec((N, 16), lambda i: (0, 0))],
        out_specs=pl.BlockSpec((_DB, N), lambda i: (i, 0)),
        out_shape=jax.ShapeDtypeStruct((N, N), f32),
    )(z, z)

    return (z_mean, z_log_std, adj)


# speedup vs baseline: 5.5353x; 1.2466x over previous
import functools

import jax
import jax.numpy as jnp
from jax import lax
from jax.experimental import pallas as pl
from jax.experimental.pallas import tpu as pltpu
from jax.experimental.pallas import tpu_sc as plsc

N = 10000
E = 160000
NC = 2
NS = 16
NW = NC * NS
CHUNK = 128
NCH0 = 64
NCH1 = 16
TOTC = NS * (NCH0 + NCH1)
EPAD = TOTC * CHUNK
NPAD = 10240
RPT = NPAD // NS

_MESH = plsc.VectorSubcoreMesh(core_axis_name="c", subcore_axis_name="s")
_SC_PARAMS = pltpu.CompilerParams(use_tc_tiling_on_sc=False)


def _chunk_range(cid, sid):
    base = jnp.where(cid == 0, sid * NCH0, NS * NCH0 + sid * NCH1)
    ngrp = jnp.where(cid == 0, NCH0 // 8, NCH1 // 8)
    return base, ngrp


@functools.partial(
    pl.kernel,
    out_type=jax.ShapeDtypeStruct((NC, NPAD, 16), jnp.float32),
    mesh=_MESH,
    scratch_types=[
        pltpu.VMEM((NCH0, CHUNK), jnp.int32),
        pltpu.VMEM((NCH0, CHUNK), jnp.int32),
        pltpu.VMEM((CHUNK, 16), jnp.float32),
        pltpu.VMEM((CHUNK, 16), jnp.float32),
        pltpu.VMEM_SHARED((NPAD, 16), jnp.float32),
        pltpu.SemaphoreType.DMA,
    ],
)
def _sc_degrees(src_hbm, dst_hbm, ones_src_hbm, ones_dst_hbm, zeros_hbm,
                out_hbm, sidx, didx, vsrc, vdst, acc, ssem):
    cid = lax.axis_index("c")
    sid = lax.axis_index("s")
    r0 = sid * RPT
    pltpu.sync_copy(zeros_hbm.at[pl.ds(r0, RPT)], acc.at[pl.ds(r0, RPT)])
    pltpu.sync_copy(ones_src_hbm, vsrc)
    pltpu.sync_copy(ones_dst_hbm, vdst)

    @pl.when(cid == 0)
    def _():
        pltpu.sync_copy(src_hbm.at[pl.ds(sid * NCH0, NCH0)], sidx)
        pltpu.sync_copy(dst_hbm.at[pl.ds(sid * NCH0, NCH0)], didx)

    @pl.when(cid == 1)
    def _():
        b = NS * NCH0 + sid * NCH1
        pltpu.sync_copy(src_hbm.at[pl.ds(b, NCH1)], sidx.at[pl.ds(0, NCH1)])
        pltpu.sync_copy(dst_hbm.at[pl.ds(b, NCH1)], didx.at[pl.ds(0, NCH1)])

    plsc.subcore_barrier()
    ngrp = jnp.where(cid == 0, NCH0 // 8, NCH1 // 8)

    def body(g, carry):
        sds = ([pltpu.async_copy(vsrc, acc.at[sidx.at[8 * g + b]],
                                 ssem, add=True) for b in range(8)] +
               [pltpu.async_copy(vdst, acc.at[didx.at[8 * g + b]],
                                 ssem, add=True) for b in range(8)])
        for d in sds:
            d.wait()
        return carry

    lax.fori_loop(0, ngrp, body, 0)
    plsc.subcore_barrier()
    pltpu.sync_copy(acc.at[pl.ds(r0, RPT)], out_hbm.at[cid, pl.ds(r0, RPT)])


def _make_seg_sum(F):
    @functools.partial(
        pl.kernel,
        out_type=jax.ShapeDtypeStruct((NC, NPAD, F), jnp.float32),
        mesh=_MESH,
        scratch_types=[
            pltpu.VMEM((NCH0, CHUNK), jnp.int32),
            pltpu.VMEM((NCH0, CHUNK), jnp.int32),
            pltpu.VMEM((8, CHUNK, F), jnp.float32),
            pltpu.VMEM_SHARED((NPAD, F), jnp.float32),
            pltpu.SemaphoreType.DMA,
            pltpu.SemaphoreType.DMA,
        ],
        compiler_params=_SC_PARAMS,
    )
    def seg_sum(t_hbm, src_hbm, dst_hbm, zeros_hbm, out_hbm,
                sidx, didx, rows, acc, gsem, ssem):
        cid = lax.axis_index("c")
        sid = lax.axis_index("s")
        r0 = sid * RPT
        pltpu.sync_copy(zeros_hbm.at[pl.ds(r0, RPT)], acc.at[pl.ds(r0, RPT)])

        @pl.when(cid == 0)
        def _():
            pltpu.sync_copy(src_hbm.at[pl.ds(sid * NCH0, NCH0)], sidx)
            pltpu.sync_copy(dst_hbm.at[pl.ds(sid * NCH0, NCH0)], didx)

        @pl.when(cid == 1)
        def _():
            b = NS * NCH0 + sid * NCH1
            pltpu.sync_copy(src_hbm.at[pl.ds(b, NCH1)], sidx.at[pl.ds(0, NCH1)])
            pltpu.sync_copy(dst_hbm.at[pl.ds(b, NCH1)], didx.at[pl.ds(0, NCH1)])

        plsc.subcore_barrier()
        ngrp = jnp.where(cid == 0, NCH0 // 8, NCH1 // 8)

        def body(g, carry):
            gds = [pltpu.async_copy(t_hbm.at[sidx.at[8 * g + b]],
                                    rows.at[b], gsem) for b in range(8)]
            for d in gds:
                d.wait()
            sds = [pltpu.async_copy(rows.at[b], acc.at[didx.at[8 * g + b]],
                                    ssem, add=True) for b in range(8)]
            for d in sds:
                d.wait()
            return carry

        lax.fori_loop(0, ngrp, body, 0)
        plsc.subcore_barrier()
        pltpu.sync_copy(acc.at[pl.ds(r0, RPT)], out_hbm.at[cid, pl.ds(r0, RPT)])

    return seg_sum


_seg_sum_64 = _make_seg_sum(64)
_seg_sum_32 = _make_seg_sum(32)


def _norm_body(degp_ref, out_ref):
    a = degp_ref[0] + degp_ref[1]
    d_src = jnp.sum(a[:, 0:8], axis=1, keepdims=True)
    d_dst = jnp.sum(a[:, 8:16], axis=1, keepdims=True)
    ns = lax.rsqrt(jnp.maximum(d_src, 1.0))
    nd = lax.rsqrt(jnp.maximum(d_dst, 1.0))
    out_ref[...] = jnp.concatenate([ns, nd], axis=1)


def _c1_body(x_ref, w_ref, nrm_ref, o_ref):
    ns = nrm_ref[:, 0:1]
    o_ref[...] = jnp.dot(x_ref[...] * ns, w_ref[...],
                         preferred_element_type=jnp.float32)


def _c2_body(agg_ref, nrm_ref, b_ref, w_ref, o_ref):
    s = agg_ref[0] + agg_ref[1]
    nd = nrm_ref[:, 1:2]
    ns = nrm_ref[:, 0:1]
    h = jnp.maximum(s * nd + b_ref[...], 0.0)
    o_ref[...] = jnp.dot(h * ns, w_ref[...], preferred_element_type=jnp.float32)


def _c3_body(agg_ref, nrm_ref, b_ref, o_ref):
    s = agg_ref[0] + agg_ref[1]
    nd = nrm_ref[:, 1:2]
    ns = nrm_ref[:, 0:1]
    o_ref[...] = (s * nd + b_ref[...]) * ns


def _c4_body(agg_ref, nrm_ref, wm_ref, bm_ref, ws_ref, bs_ref, eps_ref,
             zm_ref, zs_ref, z_ref):
    s = agg_ref[0] + agg_ref[1]
    nd = nrm_ref[:, 1:2]
    pre = s * nd
    zm = jnp.dot(pre, wm_ref[...], preferred_element_type=jnp.float32) + bm_ref[...]
    zs = jnp.dot(pre, ws_ref[...], preferred_element_type=jnp.float32) + bs_ref[...]
    zm_ref[...] = zm
    zs_ref[...] = zs
    z_ref[...] = zm + eps_ref[...] * zs


def _dec_body(zb_ref, zf_ref, o_ref):
    o_ref[...] = lax.dot_general(
        zb_ref[...], zf_ref[...],
        dimension_numbers=(((1,), (1,)), ((), ())),
        preferred_element_type=jnp.float32)


_RB = 1024
_DB = 200


def kernel(x, edge_index, eps, W1, b1, W2, b2, Wm, bm, Ws, bs):
    f32 = jnp.float32
    pad = jnp.full((EPAD - E,), N, jnp.int32)
    src = jnp.concatenate([edge_index[0], pad]).reshape(TOTC, CHUNK)
    dst = jnp.concatenate([edge_index[1], pad]).reshape(TOTC, CHUNK)

    ones_src = jnp.concatenate(
        [jnp.ones((CHUNK, 8), f32), jnp.zeros((CHUNK, 8), f32)], axis=1) / 8.0
    ones_dst = jnp.concatenate(
        [jnp.zeros((CHUNK, 8), f32), jnp.ones((CHUNK, 8), f32)], axis=1) / 8.0
    zeros_deg = jnp.zeros((NPAD, 16), f32)
    zeros_64 = jnp.zeros((NPAD, 64), f32)
    zeros_32 = jnp.zeros((NPAD, 32), f32)

    degp = _sc_degrees(src, dst, ones_src, ones_dst, zeros_deg)
    norms = pl.pallas_call(
        _norm_body,
        out_shape=jax.ShapeDtypeStruct((NPAD, 2), f32),
    )(degp)

    grid = NPAD // _RB
    row_spec = lambda F: pl.BlockSpec((_RB, F), lambda i: (i, 0))
    agg_spec = lambda F: pl.BlockSpec((NC, _RB, F), lambda i: (0, i, 0))
    full = lambda shape: pl.BlockSpec(shape, lambda i: tuple(0 for _ in shape))

    t1 = pl.pallas_call(
        _c1_body,
        grid=(grid,),
        in_specs=[row_spec(128), full((128, 64)), row_spec(2)],
        out_specs=row_spec(64),
        out_shape=jax.ShapeDtypeStruct((NPAD, 64), f32),
    )(x, W1, norms)
    agg1 = _seg_sum_64(t1, src, dst, zeros_64)

    t2 = pl.pallas_call(
        _c2_body,
        grid=(grid,),
        in_specs=[agg_spec(64), row_spec(2), full((1, 64)), full((64, 32))],
        out_specs=row_spec(32),
        out_shape=jax.ShapeDtypeStruct((NPAD, 32), f32),
    )(agg1, norms, b1.reshape(1, 64), W2)
    agg2 = _seg_sum_32(t2, src, dst, zeros_32)

    t3 = pl.pallas_call(
        _c3_body,
        grid=(grid,),
        in_specs=[agg_spec(32), row_spec(2), full((1, 32))],
        out_specs=row_spec(32),
        out_shape=jax.ShapeDtypeStruct((NPAD, 32), f32),
    )(agg2, norms, b2.reshape(1, 32))
    agg3 = _seg_sum_32(t3, src, dst, zeros_32)

    z_mean, z_log_std, z = pl.pallas_call(
        _c4_body,
        grid=(grid,),
        in_specs=[agg_spec(32), row_spec(2), full((32, 16)), full((1, 16)),
                  full((32, 16)), full((1, 16)), full((1, 16))],
        out_specs=[row_spec(16), row_spec(16), row_spec(16)],
        out_shape=[jax.ShapeDtypeStruct((N, 16), f32)] * 3,
    )(agg3, norms, Wm, bm.reshape(1, 16), Ws, bs.reshape(1, 16),
      eps.reshape(1, 16))

    adj = pl.pallas_call(
        _dec_body,
        grid=(N // _DB,),
        in_specs=[pl.BlockSpec((_DB, 16), lambda i: (i, 0)),
                  pl.BlockSpec((N, 16), lambda i: (0, 0))],
        out_specs=pl.BlockSpec((_DB, N), lambda i: (i, 0)),
        out_shape=jax.ShapeDtypeStruct((N, N), f32),
    )(z, z)

    return (z_mean, z_log_std, adj)
